# baseline (speedup 1.0000x reference)
import functools

import jax
import jax.numpy as jnp
from jax import lax
from jax.experimental import pallas as pl
from jax.experimental.pallas import tpu as pltpu
from jax.experimental.pallas import tpu_sc as plsc

N = 10000
D = 128
E = 320000

NC = 2
NS = 16
NW = NC * NS
E_PER_W = E // NW
C = 80
CHUNKS = E_PER_W // C
N_PAD = 10240
ROWS_PER_TILE = N_PAD // NS
NBUF = 4


def _make_sc_spmm():
    mesh = plsc.VectorSubcoreMesh(core_axis_name="c", subcore_axis_name="s")

    @functools.partial(
        pl.kernel,
        mesh=mesh,
        out_type=jax.ShapeDtypeStruct((NC, N_PAD, D), jnp.float32),
        scratch_types=(
            [pltpu.VMEM_SHARED((N_PAD, D), jnp.float32)]
            + [pltpu.VMEM((C,), jnp.int32) for _ in range(NBUF)]
            + [pltpu.VMEM((C,), jnp.int32) for _ in range(NBUF)]
            + [pltpu.VMEM((C,), jnp.float32) for _ in range(NBUF)]
            + [pltpu.VMEM((C, D), jnp.float32) for _ in range(NBUF)]
            + [pltpu.SemaphoreType.DMA for _ in range(5 * NBUF)]
        ),
    )
    def sc_spmm(input_hbm, col_hbm, row_hbm, val_hbm, out_hbm, acc,
                cb0, cb1, cb2, cb3, wb0, wb1, wb2, wb3, vb0, vb1, vb2, vb3,
                rb0, rb1, rb2, rb3, *sems):
        cid = lax.axis_index("c")
        sid = lax.axis_index("s")
        w = sid * NC + cid
        col_bufs = (cb0, cb1, cb2, cb3)
        row_bufs = (wb0, wb1, wb2, wb3)
        val_bufs = (vb0, vb1, vb2, vb3)
        rows_bufs = (rb0, rb1, rb2, rb3)
        semc = sems[0:NBUF]
        semr = sems[NBUF:2 * NBUF]
        semv = sems[2 * NBUF:3 * NBUF]
        semg = sems[3 * NBUF:4 * NBUF]
        sems_s = sems[4 * NBUF:5 * NBUF]
        base_w = w * E_PER_W

        def col_start(ci, s):
            pltpu.make_async_copy(
                col_hbm.at[pl.ds(base_w + ci * C, C)], col_bufs[s], semc[s]).start()

        def col_wait(ci, s):
            pltpu.make_async_copy(
                col_hbm.at[pl.ds(base_w + ci * C, C)], col_bufs[s], semc[s]).wait()

        def rv_start(ci, s):
            pltpu.make_async_copy(
                row_hbm.at[pl.ds(base_w + ci * C, C)], row_bufs[s], semr[s]).start()
            pltpu.make_async_copy(
                val_hbm.at[pl.ds(base_w + ci * C, C)], val_bufs[s], semv[s]).start()

        def rv_wait(ci, s):
            pltpu.make_async_copy(
                row_hbm.at[pl.ds(base_w + ci * C, C)], row_bufs[s], semr[s]).wait()
            pltpu.make_async_copy(
                val_hbm.at[pl.ds(base_w + ci * C, C)], val_bufs[s], semv[s]).wait()

        def gather_start(ci, b):
            pltpu.make_async_copy(
                input_hbm.at[col_bufs[b]], rows_bufs[b], semg[b]).start()

        def gather_wait(ci, b):
            pltpu.make_async_copy(
                input_hbm.at[col_bufs[b]], rows_bufs[b], semg[b]).wait()

        def scatter_start(ci, b):
            pltpu.make_async_copy(
                rows_bufs[b], acc.at[row_bufs[b]], sems_s[b]).start(add=True)

        def scatter_wait(ci, b):
            pltpu.make_async_copy(
                rows_bufs[b], acc.at[row_bufs[b]], sems_s[b]).wait()

        _dnums = lax.GatherDimensionNumbers(
            offset_dims=(), collapsed_slice_dims=(0,), start_index_map=(0,))

        def multiply(b):
            rows = rows_bufs[b]
            vb = val_bufs[b]

            @plsc.parallel_loop(0, C // 16)
            def group_body(g):
                vv = vb[pl.ds(g * 16, 16)]
                for j in range(16):
                    v = lax.gather(
                        vv, jnp.full((16, 1), j, jnp.int32), _dnums, (1,),
                        mode=lax.GatherScatterMode.PROMISE_IN_BOUNDS)
                    e = g * 16 + j
                    for d8 in range(D // 16):
                        sl = pl.ds(d8 * 16, 16)
                        rows[e, sl] = rows[e, sl] * v

        col_start(0, 0)
        rv_start(0, 0)
        col_start(1, 1)

        def zero_row(i, carry):
            for d8 in range(D // 16):
                rb2[i, pl.ds(d8 * 16, 16)] = jnp.zeros((16,), jnp.float32)
            return carry

        lax.fori_loop(0, C, zero_row, 0)
        for k in range(ROWS_PER_TILE // C):
            pltpu.make_async_copy(
                rb2, acc.at[pl.ds(sid * ROWS_PER_TILE + k * C, C)],
                sems_s[0]).start()
        for k in range(ROWS_PER_TILE // C):
            pltpu.make_async_copy(
                rb2, acc.at[pl.ds(sid * ROWS_PER_TILE + k * C, C)],
                sems_s[0]).wait()
        plsc.subcore_barrier()

        col_wait(0, 0)
        gather_start(0, 0)
        rv_start(1, 1)
        col_wait(1, 1)
        gather_start(1, 1)
        col_start(2, 2)

        def step(ci, b, drain=True, pre2=True, pre3=True):
            b2, b3 = (b + 2) % NBUF, (b + 3) % NBUF
            if drain:
                scatter_wait(ci - 2, b2)
            if pre2:
                rv_start(ci + 2, b2)
                col_wait(ci + 2, b2)
                gather_start(ci + 2, b2)
            if pre3:
                col_start(ci + 3, b3)
            gather_wait(ci, b)
            rv_wait(ci, b)
            multiply(b)
            scatter_start(ci, b)

        step(0, 0, drain=False)
        step(1, 1, drain=False)
        step(2, 2)
        step(3, 3)

        def pipe_body(cj, carry):
            for b in range(NBUF):
                step(NBUF * cj + b, b)
            return carry

        lax.fori_loop(1, 30, pipe_body, 0)

        step(120, 0)
        step(121, 1)
        step(122, 2, pre3=False)
        step(123, 3, pre2=False, pre3=False)
        step(124, 0, pre2=False, pre3=False)
        scatter_wait(CHUNKS - 2, 3)
        scatter_wait(CHUNKS - 1, 0)
        plsc.subcore_barrier()

        pltpu.sync_copy(
            acc.at[pl.ds(sid * ROWS_PER_TILE, ROWS_PER_TILE)],
            out_hbm.at[cid, pl.ds(sid * ROWS_PER_TILE, ROWS_PER_TILE)])

    return sc_spmm


_sc_spmm = _make_sc_spmm()

_BLK = 5000


def _blend_body(alpha_ref, f_ref, p0_ref, p1_ref, o_ref):
    a = alpha_ref[0]
    o_ref[...] = (1.0 - a) * (p0_ref[0] + p1_ref[0]) + a * f_ref[...]


def _blend(alpha, feature, partial):
    return pl.pallas_call(
        _blend_body,
        grid=(N // _BLK,),
        in_specs=[
            pl.BlockSpec(memory_space=pltpu.SMEM),
            pl.BlockSpec((_BLK, D), lambda i: (i, 0)),
            pl.BlockSpec((1, _BLK, D), lambda i: (0, i, 0)),
            pl.BlockSpec((1, _BLK, D), lambda i: (1, i, 0)),
        ],
        out_specs=pl.BlockSpec((_BLK, D), lambda i: (i, 0)),
        out_shape=jax.ShapeDtypeStruct((N, D), jnp.float32),
    )(alpha, feature, partial, partial)


def kernel(feature, input, adj_indices, adj_values, alpha, weight):
    del weight
    row = adj_indices[0]
    col = adj_indices[1]
    partial = _sc_spmm(input, col, row, adj_values)
    return _blend(jnp.reshape(alpha, (1,)), feature, partial)


# speedup vs baseline: 1.1561x; 1.1561x over previous
import functools

import jax
import jax.numpy as jnp
from jax import lax
from jax.experimental import pallas as pl
from jax.experimental.pallas import tpu as pltpu
from jax.experimental.pallas import tpu_sc as plsc

N = 10000
D = 128
E = 320000

NC = 2
NS = 16
NW = NC * NS
E_PER_W = E // NW
C = 80
CHUNKS = E_PER_W // C
N_PAD = 10240
ROWS_PER_TILE = N_PAD // NS
NBUF = 4


def _make_sc_spmm():
    mesh = plsc.VectorSubcoreMesh(core_axis_name="c", subcore_axis_name="s")

    @functools.partial(
        pl.kernel,
        mesh=mesh,
        out_type=jax.ShapeDtypeStruct((NC, N_PAD, D), jnp.float32),
        scratch_types=(
            [pltpu.VMEM_SHARED((N_PAD, D), jnp.float32)]
            + [pltpu.VMEM((C,), jnp.int32) for _ in range(NBUF)]
            + [pltpu.VMEM((C,), jnp.int32) for _ in range(NBUF)]
            + [pltpu.VMEM((C,), jnp.float32) for _ in range(NBUF)]
            + [pltpu.VMEM((C, D), jnp.float32) for _ in range(NBUF)]
            + [pltpu.SemaphoreType.DMA for _ in range(5 * NBUF)]
        ),
    )
    def sc_spmm(input_hbm, col_hbm, row_hbm, val_hbm, out_hbm, acc,
                cb0, cb1, cb2, cb3, wb0, wb1, wb2, wb3, vb0, vb1, vb2, vb3,
                rb0, rb1, rb2, rb3, *sems):
        cid = lax.axis_index("c")
        sid = lax.axis_index("s")
        w = sid * NC + cid
        col_bufs = (cb0, cb1, cb2, cb3)
        row_bufs = (wb0, wb1, wb2, wb3)
        val_bufs = (vb0, vb1, vb2, vb3)
        rows_bufs = (rb0, rb1, rb2, rb3)
        semc = sems[0:NBUF]
        semr = sems[NBUF:2 * NBUF]
        semv = sems[2 * NBUF:3 * NBUF]
        semg = sems[3 * NBUF:4 * NBUF]
        sems_s = sems[4 * NBUF:5 * NBUF]
        base_w = w * E_PER_W

        def col_start(ci, s):
            pltpu.make_async_copy(
                col_hbm.at[pl.ds(base_w + ci * C, C)], col_bufs[s], semc[s]).start()

        def col_wait(ci, s):
            pltpu.make_async_copy(
                col_hbm.at[pl.ds(base_w + ci * C, C)], col_bufs[s], semc[s]).wait()

        def rv_start(ci, s):
            pltpu.make_async_copy(
                row_hbm.at[pl.ds(base_w + ci * C, C)], row_bufs[s], semr[s]).start()
            pltpu.make_async_copy(
                val_hbm.at[pl.ds(base_w + ci * C, C)], val_bufs[s], semv[s]).start()

        def rv_wait(ci, s):
            pltpu.make_async_copy(
                row_hbm.at[pl.ds(base_w + ci * C, C)], row_bufs[s], semr[s]).wait()
            pltpu.make_async_copy(
                val_hbm.at[pl.ds(base_w + ci * C, C)], val_bufs[s], semv[s]).wait()

        def gather_start(ci, b):
            pltpu.make_async_copy(
                input_hbm.at[col_bufs[b]], rows_bufs[b], semg[b]).start()

        def gather_wait(ci, b):
            pltpu.make_async_copy(
                input_hbm.at[col_bufs[b]], rows_bufs[b], semg[b]).wait()

        def scatter_start(ci, b):
            pltpu.make_async_copy(
                rows_bufs[b], acc.at[row_bufs[b]], sems_s[b]).start(add=True)

        def scatter_wait(ci, b):
            pltpu.make_async_copy(
                rows_bufs[b], acc.at[row_bufs[b]], sems_s[b]).wait()

        _dnums = lax.GatherDimensionNumbers(
            offset_dims=(), collapsed_slice_dims=(0,), start_index_map=(0,))

        def multiply(b):
            rows = rows_bufs[b]
            vb = val_bufs[b]

            def group_body(g, inner):
                vv = vb[pl.ds(g * 16, 16)]
                for j in range(16):
                    v = lax.gather(
                        vv, jnp.full((16, 1), j, jnp.int32), _dnums, (1,),
                        mode=lax.GatherScatterMode.PROMISE_IN_BOUNDS)
                    e = g * 16 + j
                    for d8 in range(D // 16):
                        sl = pl.ds(d8 * 16, 16)
                        rows[e, sl] = rows[e, sl] * v
                return inner

            lax.fori_loop(0, C // 16, group_body, 0)

        col_start(0, 0)
        rv_start(0, 0)
        col_start(1, 1)

        def zero_row(i, carry):
            for d8 in range(D // 16):
                rb2[i, pl.ds(d8 * 16, 16)] = jnp.zeros((16,), jnp.float32)
            return carry

        lax.fori_loop(0, C, zero_row, 0)
        for k in range(ROWS_PER_TILE // C):
            pltpu.make_async_copy(
                rb2, acc.at[pl.ds(sid * ROWS_PER_TILE + k * C, C)],
                sems_s[0]).start()
        for k in range(ROWS_PER_TILE // C):
            pltpu.make_async_copy(
                rb2, acc.at[pl.ds(sid * ROWS_PER_TILE + k * C, C)],
                sems_s[0]).wait()
        plsc.subcore_barrier()

        col_wait(0, 0)
        gather_start(0, 0)
        rv_start(1, 1)
        col_wait(1, 1)
        gather_start(1, 1)
        col_start(2, 2)

        def step(ci, b, drain=True, pre2=True, pre3=True):
            b2, b3 = (b + 2) % NBUF, (b + 3) % NBUF
            if drain:
                scatter_wait(ci - 2, b2)
            if pre2:
                rv_start(ci + 2, b2)
                col_wait(ci + 2, b2)
                gather_start(ci + 2, b2)
            if pre3:
                col_start(ci + 3, b3)
            gather_wait(ci, b)
            rv_wait(ci, b)
            multiply(b)
            scatter_start(ci, b)

        step(0, 0, drain=False)
        step(1, 1, drain=False)
        step(2, 2)
        step(3, 3)

        def pipe_body(cj, carry):
            for b in range(NBUF):
                step(NBUF * cj + b, b)
            return carry

        lax.fori_loop(1, 30, pipe_body, 0)

        step(120, 0)
        step(121, 1)
        step(122, 2, pre3=False)
        step(123, 3, pre2=False, pre3=False)
        step(124, 0, pre2=False, pre3=False)
        scatter_wait(CHUNKS - 2, 3)
        scatter_wait(CHUNKS - 1, 0)
        plsc.subcore_barrier()

        pltpu.sync_copy(
            acc.at[pl.ds(sid * ROWS_PER_TILE, ROWS_PER_TILE)],
            out_hbm.at[cid, pl.ds(sid * ROWS_PER_TILE, ROWS_PER_TILE)])

    return sc_spmm


_sc_spmm = _make_sc_spmm()

_BLK = 5000


def _blend_body(alpha_ref, f_ref, p0_ref, p1_ref, o_ref):
    a = alpha_ref[0]
    o_ref[...] = (1.0 - a) * (p0_ref[0] + p1_ref[0]) + a * f_ref[...]


def _blend(alpha, feature, partial):
    return pl.pallas_call(
        _blend_body,
        grid=(N // _BLK,),
        in_specs=[
            pl.BlockSpec(memory_space=pltpu.SMEM),
            pl.BlockSpec((_BLK, D), lambda i: (i, 0)),
            pl.BlockSpec((1, _BLK, D), lambda i: (0, i, 0)),
            pl.BlockSpec((1, _BLK, D), lambda i: (1, i, 0)),
        ],
        out_specs=pl.BlockSpec((_BLK, D), lambda i: (i, 0)),
        out_shape=jax.ShapeDtypeStruct((N, D), jnp.float32),
    )(alpha, feature, partial, partial)


def kernel(feature, input, adj_indices, adj_values, alpha, weight):
    del weight
    row = adj_indices[0]
    col = adj_indices[1]
    partial = _sc_spmm(input, col, row, adj_values)
    return _blend(jnp.reshape(alpha, (1,)), feature, partial)
